# baseline (speedup 1.0000x reference)
import jax
import jax.numpy as jnp
from jax.experimental import pallas as pl
from jax.experimental.pallas import tpu as pltpu

B = 16
H = W = 224
SIZE = 223
K = SIZE * SIZE
NPER = 256
NR = 32
NT = NPER // NR
GRID = 3 * NT
KB = 25088
KPAD = 2 * KB


def _fused_kernel(cw_ref, x_ref, wr_ref, wg_ref, wb_ref, bias_ref,
                  out_ref, flats_ref, acc_ref):
    i = pl.program_id(0)
    k = pl.program_id(1)

    @pl.when((i == 0) & (k == 0))
    def _zero_tail():
        flats_ref[:, :, K:] = jnp.zeros((3, B, KPAD - K), jnp.float32)

    for c in range(3):
        @pl.when((i == c * NT) & (k == 0))
        def _conv(c=c):
            w00 = cw_ref[c, 0]
            w01 = cw_ref[c, 1]
            w10 = cw_ref[c, 2]
            w11 = cw_ref[c, 3]
            xs = x_ref[c]
            y = jnp.tanh(
                w00 * xs[:, :SIZE, :SIZE]
                + w01 * xs[:, :SIZE, 1:]
                + w10 * xs[:, 1:, :SIZE]
                + w11 * xs[:, 1:, 1:]
            )
            for r in range(SIZE):
                flats_ref[c, :, r * SIZE:(r + 1) * SIZE] = y[:, r, :]

    for c, wref in enumerate((wr_ref, wg_ref, wb_ref)):
        @pl.when((i >= c * NT) & (i < (c + 1) * NT))
        def _mm(c=c, wref=wref):
            w = wref[...]

            @pl.when(k == 0)
            def _first():
                f = flats_ref[c, :, :KB]
                acc_ref[...] = jax.lax.dot_general(
                    f, w, (((1,), (1,)), ((), ())),
                    preferred_element_type=jnp.float32)

            @pl.when(k == 1)
            def _second():
                cols = KB + jax.lax.broadcasted_iota(jnp.int32, (NR, KB), 1)
                wm = jnp.where(cols < K, w, 0.0)
                f = flats_ref[c, :, KB:]
                y = jax.lax.dot_general(
                    f, wm, (((1,), (1,)), ((), ())),
                    preferred_element_type=jnp.float32)
                out_ref[0] = jnp.tanh(acc_ref[...] + y + bias_ref[0])


def _fused(x, cw, fw_r, fw_g, fw_b, bias, interpret=False):
    return pl.pallas_call(
        _fused_kernel,
        grid=(GRID, 2),
        in_specs=[
            pl.BlockSpec(memory_space=pltpu.SMEM),
            pl.BlockSpec((3, B, H, W), lambda i, k: (0, 0, 0, 0)),
            pl.BlockSpec((NR, KB), lambda i, k: (jnp.minimum(i, NT - 1), k)),
            pl.BlockSpec((NR, KB), lambda i, k: (jnp.clip(i - NT, 0, NT - 1), k)),
            pl.BlockSpec((NR, KB), lambda i, k: (jnp.clip(i - 2 * NT, 0, NT - 1), k)),
            pl.BlockSpec((1, 1, NR), lambda i, k: (i, 0, 0)),
        ],
        out_specs=pl.BlockSpec((1, B, NR), lambda i, k: (i, 0, 0)),
        out_shape=jax.ShapeDtypeStruct((GRID, B, NR), jnp.float32),
        scratch_shapes=[pltpu.VMEM((3, B, KPAD), jnp.float32),
                        pltpu.VMEM((B, NR), jnp.float32)],
        compiler_params=pltpu.CompilerParams(
            dimension_semantics=("arbitrary", "arbitrary")),
        interpret=interpret,
    )(cw, x, fw_r, fw_g, fw_b, bias)


def kernel(x, w_red, w_green, w_blue, fc_red_w, fc_red_b,
           fc_green_w, fc_green_b, fc_blue_w, fc_blue_b,
           interpret=False):
    cw = jnp.stack([w_red.reshape(4), w_green.reshape(4), w_blue.reshape(4)])
    bias = jnp.concatenate([fc_red_b, fc_green_b, fc_blue_b]).reshape(GRID, 1, NR)
    tiles = _fused(x, cw, fc_red_w, fc_green_w, fc_blue_w, bias,
                   interpret=interpret)
    return tiles.transpose(1, 0, 2).reshape(B, 3 * NPER)


# speedup vs baseline: 1.4738x; 1.4738x over previous
import jax
import jax.numpy as jnp
from jax.experimental import pallas as pl
from jax.experimental.pallas import tpu as pltpu

B = 16
H = W = 224
SIZE = 223
K = SIZE * SIZE
NPER = 256
NR = 32
NT = NPER // NR
GRID = 3 * NT
NBUF = 4
SPLIT = 2
RS = NR // SPLIT


def _fused_kernel(cw_ref, x_ref, wr_ref, wg_ref, wb_ref, bias_ref,
                  out_ref, flats_ref, wbuf_ref, sems):
    t = pl.program_id(0)
    wrefs = (wr_ref, wg_ref, wb_ref)

    def issue(tile, slot):
        c = tile // NT
        j = tile % NT
        for ci in range(3):
            @pl.when(c == ci)
            def _(ci=ci):
                for s in range(SPLIT):
                    pltpu.make_async_copy(
                        wrefs[ci].at[pl.ds(j * NR + s * RS, RS), :],
                        wbuf_ref.at[slot, pl.ds(s * RS, RS), :],
                        sems.at[slot, s]).start()

    @pl.when(t == 0)
    def _warmup():
        for tt in range(NBUF):
            issue(tt, tt)

    for c in range(3):
        @pl.when(t == c * NT)
        def _conv(c=c):
            w00 = cw_ref[c, 0]
            w01 = cw_ref[c, 1]
            w10 = cw_ref[c, 2]
            w11 = cw_ref[c, 3]
            xs = x_ref[c]
            y = jnp.tanh(
                w00 * xs[:, :SIZE, :SIZE]
                + w01 * xs[:, :SIZE, 1:]
                + w10 * xs[:, 1:, :SIZE]
                + w11 * xs[:, 1:, 1:]
            )
            for r in range(SIZE):
                flats_ref[c, :, r * SIZE:(r + 1) * SIZE] = y[:, r, :]

    slot = t % NBUF
    j = t % NT
    for ci in range(3):
        @pl.when(t // NT == ci)
        def _mm(ci=ci):
            for s in range(SPLIT):
                pltpu.make_async_copy(
                    wrefs[ci].at[pl.ds(j * NR + s * RS, RS), :],
                    wbuf_ref.at[slot, pl.ds(s * RS, RS), :],
                    sems.at[slot, s]).wait()
            f = flats_ref[ci]
            w = wbuf_ref[slot]
            y = jax.lax.dot_general(
                f, w, (((1,), (1,)), ((), ())),
                preferred_element_type=jnp.float32)
            out_ref[t] = jnp.tanh(y + bias_ref[t])

    nxt = t + NBUF

    @pl.when(nxt < GRID)
    def _refill():
        issue(nxt, slot)


def _fused(x, cw, fw_r, fw_g, fw_b, bias, interpret=False):
    return pl.pallas_call(
        _fused_kernel,
        grid=(GRID,),
        in_specs=[
            pl.BlockSpec(memory_space=pltpu.SMEM),
            pl.BlockSpec((3, B, H, W), lambda i: (0, 0, 0, 0)),
            pl.BlockSpec(memory_space=pltpu.MemorySpace.HBM),
            pl.BlockSpec(memory_space=pltpu.MemorySpace.HBM),
            pl.BlockSpec(memory_space=pltpu.MemorySpace.HBM),
            pl.BlockSpec((GRID, 1, NR), lambda i: (0, 0, 0)),
        ],
        out_specs=pl.BlockSpec((GRID, B, NR), lambda i: (0, 0, 0)),
        out_shape=jax.ShapeDtypeStruct((GRID, B, NR), jnp.float32),
        scratch_shapes=[pltpu.VMEM((3, B, K), jnp.float32),
                        pltpu.VMEM((NBUF, NR, K), jnp.float32),
                        pltpu.SemaphoreType.DMA((NBUF, SPLIT))],
        compiler_params=pltpu.CompilerParams(
            dimension_semantics=("arbitrary",)),
        interpret=interpret,
    )(cw, x, fw_r, fw_g, fw_b, bias)


def kernel(x, w_red, w_green, w_blue, fc_red_w, fc_red_b,
           fc_green_w, fc_green_b, fc_blue_w, fc_blue_b,
           interpret=False):
    cw = jnp.stack([w_red.reshape(4), w_green.reshape(4), w_blue.reshape(4)])
    bias = jnp.concatenate([fc_red_b, fc_green_b, fc_blue_b]).reshape(GRID, 1, NR)
    tiles = _fused(x, cw, fc_red_w, fc_green_w, fc_blue_w, bias,
                   interpret=interpret)
    return tiles.transpose(1, 0, 2).reshape(B, 3 * NPER)
